# baseline (speedup 1.0000x reference)
import functools

import jax
import jax.numpy as jnp
from jax.experimental import pallas as pl
from jax.experimental.pallas import tpu as pltpu

K = 8
CHUNK = 2048
DB = 1024

_SORT8 = ((0, 1), (2, 3), (4, 5), (6, 7), (0, 2), (1, 3), (4, 6), (5, 7),
          (1, 2), (5, 6), (0, 4), (1, 5), (2, 6), (3, 7), (2, 4), (3, 5),
          (1, 2), (3, 4), (5, 6))
_BITONIC8 = ((0, 4), (1, 5), (2, 6), (3, 7), (0, 2), (1, 3), (4, 6), (5, 7),
             (0, 1), (2, 3), (4, 5), (6, 7))


def _ce(items, pairs):
    for i, j in pairs:
        a, b = items[i], items[j]
        items[i] = jnp.maximum(a, b)
        items[j] = jnp.minimum(a, b)


def _kmax_kernel(x_ref, o_ref, t_ref, *, n_chunks):
    c = pl.program_id(2)

    @pl.when(c == 0)
    def _init():
        t_ref[...] = jnp.full(t_ref.shape, -jnp.inf, jnp.float32)

    T = [t_ref[8 * k:8 * k + 8, :] for k in range(2 * K)]

    def leaf16(base):
        a = [x_ref[0, pl.ds(base + j * 8, 8), :] for j in range(8)]
        b = [x_ref[0, pl.ds(base + 64 + j * 8, 8), :] for j in range(8)]
        _ce(a, _SORT8)
        _ce(b, _SORT8)
        s = [jnp.maximum(a[i], b[7 - i]) for i in range(8)]
        _ce(s, _BITONIC8)
        return s

    def group_body(g, Ts):
        out = []
        for h in range(2):
            s = leaf16(g * 256 + h * 128)
            m = [jnp.maximum(Ts[h * K + i], s[7 - i]) for i in range(8)]
            _ce(m, _BITONIC8)
            out.extend(m)
        return tuple(out)

    T = jax.lax.fori_loop(0, CHUNK // 256, group_body, tuple(T), unroll=4)
    for k in range(2 * K):
        t_ref[8 * k:8 * k + 8, :] = T[k]

    @pl.when(c == n_chunks - 1)
    def _finalize():
        cand = t_ref[...]
        iota = jax.lax.broadcasted_iota(jnp.int32, cand.shape, 0)
        for j in range(K):
            m = jnp.max(cand, axis=0, keepdims=True)
            idx = jnp.min(jnp.where(cand == m, iota, cand.shape[0]), axis=0,
                          keepdims=True)
            cand = jnp.where(iota == idx, -jnp.inf, cand)
            o_ref[0, j, :] = m[0]


def kernel(inputs):
    B, S, D = inputs.shape
    n_chunks = S // CHUNK
    out = pl.pallas_call(
        functools.partial(_kmax_kernel, n_chunks=n_chunks),
        grid=(B, D // DB, n_chunks),
        in_specs=[pl.BlockSpec((1, CHUNK, DB), lambda b, d, c: (b, c, d))],
        out_specs=pl.BlockSpec((1, K, DB), lambda b, d, c: (b, 0, d)),
        out_shape=jax.ShapeDtypeStruct((B, K, D), jnp.float32),
        scratch_shapes=[pltpu.VMEM((16 * K, DB), jnp.float32)],
    )(inputs)
    return out.transpose(0, 2, 1).reshape(B, D * K)


# speedup vs baseline: 1.1667x; 1.1667x over previous
import functools

import jax
import jax.numpy as jnp
from jax.experimental import pallas as pl
from jax.experimental.pallas import tpu as pltpu

K = 8
CHUNK = 2048
DB = 1024

_SORT8 = ((0, 1), (2, 3), (4, 5), (6, 7), (0, 2), (1, 3), (4, 6), (5, 7),
          (1, 2), (5, 6), (0, 4), (1, 5), (2, 6), (3, 7), (2, 4), (3, 5),
          (1, 2), (3, 4), (5, 6))
_BITONIC8 = ((0, 4), (1, 5), (2, 6), (3, 7), (0, 2), (1, 3), (4, 6), (5, 7),
             (0, 1), (2, 3), (4, 5), (6, 7))


def _ce(items, pairs):
    for i, j in pairs:
        a, b = items[i], items[j]
        items[i] = jnp.maximum(a, b)
        items[j] = jnp.minimum(a, b)


def _kmax_kernel(x_ref, o_ref, t_ref, *, n_chunks):
    c = pl.program_id(2)

    @pl.when(c == 0)
    def _init():
        t_ref[...] = jnp.full(t_ref.shape, -jnp.inf, jnp.float32)

    T = [t_ref[8 * k:8 * k + 8, :] for k in range(K)]

    def group_body(g, Ts):
        a = [x_ref[0, pl.ds(g * 128 + j * 8, 8), :] for j in range(8)]
        b = [x_ref[0, pl.ds(g * 128 + 64 + j * 8, 8), :] for j in range(8)]
        _ce(a, _SORT8)
        _ce(b, _SORT8)
        s = [jnp.maximum(a[i], b[7 - i]) for i in range(8)]
        _ce(s, _BITONIC8)
        merged = [jnp.maximum(Ts[i], s[7 - i]) for i in range(8)]
        _ce(merged, _BITONIC8)
        return tuple(merged)

    T = jax.lax.fori_loop(0, CHUNK // 128, group_body, tuple(T), unroll=16)
    for k in range(K):
        t_ref[8 * k:8 * k + 8, :] = T[k]

    @pl.when(c == n_chunks - 1)
    def _finalize():
        cand = t_ref[...]
        iota = jax.lax.broadcasted_iota(jnp.int32, cand.shape, 0)
        for j in range(K):
            m = jnp.max(cand, axis=0, keepdims=True)
            idx = jnp.min(jnp.where(cand == m, iota, cand.shape[0]), axis=0,
                          keepdims=True)
            cand = jnp.where(iota == idx, -jnp.inf, cand)
            o_ref[0, j, :] = m[0]


def kernel(inputs):
    B, S, D = inputs.shape
    n_chunks = S // CHUNK
    out = pl.pallas_call(
        functools.partial(_kmax_kernel, n_chunks=n_chunks),
        grid=(B, D // DB, n_chunks),
        in_specs=[pl.BlockSpec((1, CHUNK, DB), lambda b, d, c: (b, c, d))],
        out_specs=pl.BlockSpec((1, K, DB), lambda b, d, c: (b, 0, d)),
        out_shape=jax.ShapeDtypeStruct((B, K, D), jnp.float32),
        scratch_shapes=[pltpu.VMEM((8 * K, DB), jnp.float32)],
    )(inputs)
    return out.transpose(0, 2, 1).reshape(B, D * K)


# speedup vs baseline: 1.2290x; 1.0534x over previous
import functools

import jax
import jax.numpy as jnp
from jax.experimental import pallas as pl
from jax.experimental.pallas import tpu as pltpu

K = 8
CHUNK = 4096
DB = 1024

_SORT8 = ((0, 1), (2, 3), (4, 5), (6, 7), (0, 2), (1, 3), (4, 6), (5, 7),
          (1, 2), (5, 6), (0, 4), (1, 5), (2, 6), (3, 7), (2, 4), (3, 5),
          (1, 2), (3, 4), (5, 6))
_BITONIC8 = ((0, 4), (1, 5), (2, 6), (3, 7), (0, 2), (1, 3), (4, 6), (5, 7),
             (0, 1), (2, 3), (4, 5), (6, 7))


def _ce(items, pairs):
    for i, j in pairs:
        a, b = items[i], items[j]
        items[i] = jnp.maximum(a, b)
        items[j] = jnp.minimum(a, b)


def _kmax_kernel(x_ref, o_ref, t_ref, *, n_chunks):
    c = pl.program_id(2)

    @pl.when(c == 0)
    def _init():
        t_ref[...] = jnp.full(t_ref.shape, -jnp.inf, jnp.float32)

    T = [t_ref[8 * k:8 * k + 8, :] for k in range(K)]

    def group_body(g, Ts):
        a = [x_ref[0, pl.ds(g * 128 + j * 8, 8), :] for j in range(8)]
        b = [x_ref[0, pl.ds(g * 128 + 64 + j * 8, 8), :] for j in range(8)]
        _ce(a, _SORT8)
        _ce(b, _SORT8)
        s = [jnp.maximum(a[i], b[7 - i]) for i in range(8)]
        _ce(s, _BITONIC8)
        merged = [jnp.maximum(Ts[i], s[7 - i]) for i in range(8)]
        _ce(merged, _BITONIC8)
        return tuple(merged)

    T = jax.lax.fori_loop(0, CHUNK // 128, group_body, tuple(T), unroll=32)
    for k in range(K):
        t_ref[8 * k:8 * k + 8, :] = T[k]

    @pl.when(c == n_chunks - 1)
    def _finalize():
        cand = t_ref[...]
        iota = jax.lax.broadcasted_iota(jnp.int32, cand.shape, 0)
        for j in range(K):
            m = jnp.max(cand, axis=0, keepdims=True)
            idx = jnp.min(jnp.where(cand == m, iota, cand.shape[0]), axis=0,
                          keepdims=True)
            cand = jnp.where(iota == idx, -jnp.inf, cand)
            o_ref[0, j, :] = m[0]


def kernel(inputs):
    B, S, D = inputs.shape
    n_chunks = S // CHUNK
    out = pl.pallas_call(
        functools.partial(_kmax_kernel, n_chunks=n_chunks),
        grid=(B, D // DB, n_chunks),
        in_specs=[pl.BlockSpec((1, CHUNK, DB), lambda b, d, c: (b, c, d))],
        out_specs=pl.BlockSpec((1, K, DB), lambda b, d, c: (b, 0, d)),
        out_shape=jax.ShapeDtypeStruct((B, K, D), jnp.float32),
        scratch_shapes=[pltpu.VMEM((8 * K, DB), jnp.float32)],
    )(inputs)
    return out.transpose(0, 2, 1).reshape(B, D * K)
